# Initial kernel scaffold; baseline (speedup 1.0000x reference)
import jax
import jax.numpy as jnp
from jax.experimental import pallas as pl


def kernel(pc, feat, new_pc, W1, b1, g1, beta1, W2, b2, g2, beta2, W3, b3, g3, beta3):
    raise NotImplementedError("write your pallas kernel here")


# speedup vs baseline: 21.8894x; 21.8894x over previous
import functools

import jax
import jax.numpy as jnp
from jax import lax
from jax.experimental import pallas as pl
from jax.experimental.pallas import tpu as pltpu
from jax.experimental.pallas import tpu_sc as plsc

_B, _N, _M, _K = 4, 8192, 2048, 64
_INFEA = 16
_C0 = _INFEA + 3
_H1, _H2, _H3 = 32, 32, 64
_DIST2 = 0.4 * 0.4
_EPS = 1e-5

_MB = 128
_MBG = 128
_T = _MBG * _K
_D = 128
_NC, _NS = 2, 16
_CH = 512


def _ballquery_kernel(pc_ref, npc_ref, idx_ref, num_ref):
    b = pl.program_id(0)
    i = pl.program_id(1)
    pc = pc_ref[0]
    npc = npc_ref[0]
    cn = jnp.sum(npc * npc, axis=0)[:, None]
    pn = jnp.sum(pc * pc, axis=0)[None, :]
    cp = jnp.dot(npc.T, pc, preferred_element_type=jnp.float32)
    d2 = cn + pn - 2.0 * cp
    mask = d2 < _DIST2
    maskf = mask.astype(jnp.float32)
    S = 64
    C = _N // S
    iota_n = jax.lax.broadcasted_iota(jnp.int32, (_MB, _N), 1)
    lane_mod = iota_n % S
    x = maskf
    for s in (1, 2, 4, 8, 16, 32):
        sh = jnp.concatenate(
            [jnp.zeros((_MB, s), jnp.float32), x[:, :_N - s]], axis=1)
        x = x + jnp.where(lane_mod >= s, sh, 0.0)
    en = jax.lax.broadcasted_iota(jnp.int32, (_N, C), 0)
    ec = jax.lax.broadcasted_iota(jnp.int32, (_N, C), 1)
    seg = (en // S == ec).astype(jnp.float32)
    tots = jnp.dot(maskf, seg, preferred_element_type=jnp.float32)
    c0 = jax.lax.broadcasted_iota(jnp.int32, (C, C), 0)
    c1 = jax.lax.broadcasted_iota(jnp.int32, (C, C), 1)
    lower_exc = (c0 < c1).astype(jnp.float32)
    offs = jnp.dot(tots, lower_exc, preferred_element_type=jnp.float32)
    offs_full = jnp.dot(offs, seg.T, preferred_element_type=jnp.float32)
    c = (x + offs_full).astype(jnp.int32)
    cumincl = (offs + tots).astype(jnp.int32)
    counts = cumincl[:, C - 1]
    kp1 = jax.lax.broadcasted_iota(jnp.int32, (_MB, _K), 1) + 1
    lo = jnp.zeros((_MB, _K), jnp.int32)
    hi = jnp.full((_MB, _K), C, jnp.int32)
    for _ in range(8):
        mid = (lo + hi) // 2
        v = jnp.take_along_axis(cumincl, jnp.minimum(mid, C - 1), axis=1)
        ge = v >= kp1
        hi = jnp.where(ge, mid, hi)
        lo = jnp.where(ge, lo, mid + 1)
    cid = jnp.minimum(lo, C - 1)
    vch = cid // 2
    base = (cid % 2) * S
    res = jnp.zeros((_MB, _K), jnp.int32)
    for cc in range(_N // 128):
        ccol = c[:, cc * 128:(cc + 1) * 128]
        lo = jnp.zeros((_MB, _K), jnp.int32)
        hi = jnp.full((_MB, _K), S, jnp.int32)
        for _ in range(7):
            mid = (lo + hi) // 2
            v = jnp.take_along_axis(ccol, base + jnp.minimum(mid, S - 1),
                                    axis=1)
            ge = v >= kp1
            hi = jnp.where(ge, mid, hi)
            lo = jnp.where(ge, lo, mid + 1)
        res = jnp.where(vch == cc, cid * S + lo, res)
    valid = kp1 <= counts[:, None]
    idx_ref[0] = jnp.where(valid, res, 0) + b * _N
    num_ref[0, 0, pl.ds(i * _MB, _MB)] = jnp.minimum(counts, _K)


def _sc_gather(tbl, gidx):
    nw = _NC * _NS
    bt = _B * _M * _K
    bpw = bt // nw
    mesh = plsc.VectorSubcoreMesh(core_axis_name="c", subcore_axis_name="s")

    @functools.partial(
        pl.kernel, mesh=mesh,
        out_type=jax.ShapeDtypeStruct((bt, _D), jnp.float32),
        scratch_types=[
            pltpu.VMEM((_CH,), jnp.int32),
            pltpu.VMEM((_CH, _D), jnp.float32),
            pltpu.SemaphoreType.DMA,
        ],
    )
    def k(tbl_hbm, idx_hbm, out_hbm, idx_v, rows_v, sem):
        wid = lax.axis_index("s") * _NC + lax.axis_index("c")
        base = wid * bpw
        for t in range(bpw // _CH):
            off = base + t * _CH
            pltpu.sync_copy(idx_hbm.at[pl.ds(off, _CH)], idx_v)
            pltpu.async_copy(tbl_hbm.at[idx_v], rows_v, sem).wait()
            pltpu.sync_copy(rows_v, out_hbm.at[pl.ds(off, _CH)])

    return k(tbl, gidx)


def _expand_mt():
    t_i = jax.lax.broadcasted_iota(jnp.int32, (_T, _MBG), 0)
    m_i = jax.lax.broadcasted_iota(jnp.int32, (_T, _MBG), 1)
    return (t_i // _K == m_i).astype(jnp.float32)


def _centered(g_ref, npc_ref):
    g = g_ref[0]
    npc = npc_ref[0]
    npc32 = jnp.concatenate(
        [npc.T, jnp.zeros((_MBG, _D - 3), jnp.float32)], axis=1)
    rep = jnp.dot(_expand_mt(), npc32, preferred_element_type=jnp.float32)
    return g - rep


def _layer(x_rowmajor_or_ch, a_ref, c_ref, contract):
    a = a_ref[...]
    y = jax.lax.dot_general(
        a, x_rowmajor_or_ch, (((1,), (contract,)), ((), ())),
        preferred_element_type=jnp.float32)
    return jnp.maximum(y + c_ref[...][:, None], 0.0)


def _moments_kernel(g_ref, npc_ref, a1_ref, c1_ref, a2_ref, c2_ref,
                    g_out, *, stage):
    b = pl.program_id(0)
    i = pl.program_id(1)
    xc = _centered(g_ref, npc_ref)
    if stage == 0:
        gm = jax.lax.dot_general(xc, xc, (((0,), (0,)), ((), ())),
                                 preferred_element_type=jnp.float32)
    else:
        z = _layer(xc, a1_ref, c1_ref, 1)
        if stage == 2:
            z = _layer(z, a2_ref, c2_ref, 0)
        zt = jnp.concatenate([z, jnp.ones((1, _T), jnp.float32)], axis=0)
        gm = jax.lax.dot_general(zt, zt, (((1,), (1,)), ((), ())),
                                 preferred_element_type=jnp.float32)

    @pl.when(jnp.logical_and(b == 0, i == 0))
    def _():
        g_out[...] = jnp.zeros_like(g_out)

    g_out[...] += gm


def _final_kernel(g_ref, npc_ref, num_ref, a1_ref, c1_ref, a2_ref, c2_ref,
                  a3_ref, c3_ref, out_ref):
    i = pl.program_id(1)
    xc = _centered(g_ref, npc_ref)
    z = _layer(xc, a1_ref, c1_ref, 1)
    z = _layer(z, a2_ref, c2_ref, 0)
    z = _layer(z, a3_ref, c3_ref, 0)
    valid = (num_ref[0, 0, pl.ds(i * _MBG, _MBG)] > 0).astype(jnp.float32)
    rep = jnp.dot(valid[None, :], _expand_mt().T,
                  preferred_element_type=jnp.float32)
    out_ref[0] = z * rep


def _affine(G, W, b, g, beta, c_in):
    cnt = G[c_in, c_in]
    mu = G[:c_in, c_in] / cnt
    cov = G[:c_in, :c_in] / cnt - jnp.outer(mu, mu)
    muy = W @ mu + b
    vary = jnp.einsum('oc,cd,od->o', W, cov, W)
    a = g * jax.lax.rsqrt(vary + _EPS)
    return a[:, None] * W, beta + a * (b - muy)


def kernel(pc, feat, new_pc, W1, b1, g1, beta1, W2, b2, g2, beta2,
           W3, b3, g3, beta3):
    idx, num = pl.pallas_call(
        _ballquery_kernel,
        grid=(_B, _M // _MB),
        in_specs=[
            pl.BlockSpec((1, 3, _N), lambda b, i: (b, 0, 0)),
            pl.BlockSpec((1, 3, _MB), lambda b, i: (b, 0, i)),
        ],
        out_specs=[
            pl.BlockSpec((1, _MB, _K), lambda b, i: (b, i, 0)),
            pl.BlockSpec((1, 1, _M), lambda b, i: (b, 0, 0)),
        ],
        out_shape=[
            jax.ShapeDtypeStruct((_B, _M, _K), jnp.int32),
            jax.ShapeDtypeStruct((_B, 1, _M), jnp.int32),
        ],
    )(pc, new_pc)

    tbl = jnp.concatenate(
        [jnp.transpose(pc, (0, 2, 1)), jnp.transpose(feat, (0, 2, 1)),
         jnp.ones((_B, _N, 1), jnp.float32),
         jnp.zeros((_B, _N, _D - _C0 - 1), jnp.float32)],
        axis=2).reshape(_B * _N, _D)
    gx = _sc_gather(tbl, idx.reshape(_B * _M * _K)).reshape(_B, _M * _K, _D)

    grid_g = (_B, _M // _MBG)
    base_specs = [
        pl.BlockSpec((1, _T, _D), lambda b, i: (b, i, 0)),
        pl.BlockSpec((1, 3, _MBG), lambda b, i: (b, 0, i)),
    ]
    full = lambda s: pl.BlockSpec(s, lambda b, i: tuple(0 for _ in s))

    def moments(stage, c_out, a1, c1, a2, c2):
        return pl.pallas_call(
            functools.partial(_moments_kernel, stage=stage),
            grid=grid_g,
            in_specs=base_specs + [full(a1.shape), full(c1.shape),
                                   full(a2.shape), full(c2.shape)],
            out_specs=pl.BlockSpec((c_out, c_out), lambda b, i: (0, 0)),
            out_shape=jax.ShapeDtypeStruct((c_out, c_out), jnp.float32),
        )(gx, new_pc, a1, c1, a2, c2)

    dummy = jnp.zeros((1, 1), jnp.float32), jnp.zeros((1,), jnp.float32)
    G0 = moments(0, _D, *dummy, *dummy)
    A1, C1 = _affine(G0, W1, b1, g1, beta1, _C0)
    A1p = jnp.concatenate(
        [A1, jnp.zeros((_H1, _D - _C0), jnp.float32)], axis=1)
    G1 = moments(1, _H1 + 1, A1p, C1, *dummy)
    A2, C2 = _affine(G1, W2, b2, g2, beta2, _H1)
    G2 = moments(2, _H2 + 1, A1p, C1, A2, C2)
    A3, C3 = _affine(G2, W3, b3, g3, beta3, _H2)

    out = pl.pallas_call(
        _final_kernel,
        grid=grid_g,
        in_specs=base_specs + [
            pl.BlockSpec((1, 1, _M), lambda b, i: (b, 0, 0)),
            full(A1p.shape), full(C1.shape), full(A2.shape), full(C2.shape),
            full(A3.shape), full(C3.shape),
        ],
        out_specs=pl.BlockSpec((1, _H3, _T), lambda b, i: (b, 0, i)),
        out_shape=jax.ShapeDtypeStruct((_B, _H3, _M * _K), jnp.float32),
    )(gx, new_pc, num, A1p, C1, A2, C2, A3, C3)
    return out.reshape(_B, _H3, _M, _K)
